# baseline (speedup 1.0000x reference)
import functools

import jax
import jax.numpy as jnp
from jax import lax
from jax.experimental import pallas as pl
from jax.experimental.pallas import tpu as pltpu
from jax.experimental.pallas import tpu_sc as plsc

_V = 100000
_D = 64
_N = 204800
_VPAD = 102400
_STRIPE = 6400
_ROWS = 50
_LANE = 128
_NTILES = 32
_VB = 5000
_NB = _V // _VB


def _sc_counts(story3d):
    mesh = plsc.VectorSubcoreMesh(core_axis_name="c", subcore_axis_name="s")

    @functools.partial(
        pl.kernel,
        out_type=jax.ShapeDtypeStruct((2, _VPAD), jnp.float32),
        mesh=mesh,
        scratch_types=[
            pltpu.VMEM((_ROWS, _LANE), jnp.int32),
            pltpu.VMEM((_STRIPE,), jnp.float32),
            pltpu.VMEM((_LANE,), jnp.float32),
            pltpu.VMEM_SHARED((_VPAD,), jnp.float32),
        ],
    )
    def k(story_hbm, out_hbm, idx_v, zeros_v, ones_v, counts_sh):
        cid = lax.axis_index("c")
        sid = lax.axis_index("s")
        tile = sid * 2 + cid

        @pl.loop(0, _STRIPE, step=16)
        def _(i):
            zeros_v[pl.ds(i, 16)] = jnp.zeros((16,), jnp.float32)

        @pl.loop(0, _LANE, step=16)
        def _(i):
            ones_v[pl.ds(i, 16)] = jnp.ones((16,), jnp.float32)

        pltpu.sync_copy(zeros_v, counts_sh.at[pl.ds(sid * _STRIPE, _STRIPE)])
        pltpu.sync_copy(story_hbm.at[tile], idx_v)
        plsc.subcore_barrier()

        @pl.loop(0, _ROWS)
        def _(j):
            pltpu.sync_copy(ones_v, counts_sh.at[idx_v.at[j]], add=True)

        plsc.subcore_barrier()
        pltpu.sync_copy(
            counts_sh.at[pl.ds(sid * _STRIPE, _STRIPE)],
            out_hbm.at[cid, pl.ds(sid * _STRIPE, _STRIPE)],
        )

    return k(story3d)


def _tc_body(n_ref, c1_ref, c2_ref, c3_ref, out_ref,
             u_ref, onum_ref, stash_ref, m_ref, z_ref):
    p = pl.program_id(0)
    i = pl.program_id(1)

    @pl.when((p == 0) & (i == 0))
    def _():
        u_ref[...] = jnp.zeros_like(u_ref)

    @pl.when(i == 0)
    def _():
        onum_ref[...] = jnp.zeros_like(onum_ref)
        z_ref[0] = 0.0
        m_ref[0] = -jnp.inf

    n = n_ref[0, 0, :][None, :]

    def online_step(l, c):
        m_old = m_ref[0]
        bm = jnp.maximum(m_old, jnp.max(l))
        scale = jnp.exp(m_old - bm)
        e = n * jnp.exp(l - bm)
        z_ref[0] = z_ref[0] * scale + jnp.sum(e)
        onum_ref[...] = onum_ref[...] * scale + lax.dot_general(
            e, c, (((1,), (0,)), ((), ())),
            preferred_element_type=jnp.float32)
        m_ref[0] = bm

    def l_row():
        return lax.dot_general(
            u_ref[...].astype(jnp.bfloat16), stash_ref[i],
            (((1,), (1,)), ((), ())),
            preferred_element_type=jnp.float32)

    @pl.when(p == 0)
    def _():
        c = c1_ref[...]
        stash_ref[i] = c.astype(jnp.bfloat16)
        online_step(jnp.zeros((1, _VB), jnp.float32), c)

    @pl.when(p == 1)
    def _():
        l = l_row()
        c = c2_ref[...]
        stash_ref[i] = c.astype(jnp.bfloat16)
        online_step(l, c)

    @pl.when(p == 2)
    def _():
        online_step(l_row(), c3_ref[...])

    @pl.when(i == _NB - 1)
    def _():
        u_new = u_ref[...] + onum_ref[...] / z_ref[0]
        u_ref[...] = u_new

        @pl.when(p == 2)
        def _():
            out_ref[...] = u_new


def _tc_hops(counts3d, C1, C2, C3, interpret=False):
    return pl.pallas_call(
        _tc_body,
        grid=(3, _NB),
        in_specs=[
            pl.BlockSpec((1, 1, _VB), lambda p, i: (i, 0, 0)),
            pl.BlockSpec((_VB, _D),
                         lambda p, i: (jnp.where(p == 0, i, _NB - 1), 0)),
            pl.BlockSpec((_VB, _D),
                         lambda p, i: (jnp.where(p == 1, i,
                                                 jnp.where(p < 1, 0, _NB - 1)), 0)),
            pl.BlockSpec((_VB, _D), lambda p, i: (jnp.where(p == 2, i, 0), 0)),
        ],
        out_specs=pl.BlockSpec((1, _D), lambda p, i: (0, 0)),
        out_shape=jax.ShapeDtypeStruct((1, _D), jnp.float32),
        scratch_shapes=[
            pltpu.VMEM((1, _D), jnp.float32),
            pltpu.VMEM((1, _D), jnp.float32),
            pltpu.VMEM((_NB, _VB, _D), jnp.bfloat16),
            pltpu.SMEM((1,), jnp.float32),
            pltpu.SMEM((1,), jnp.float32),
        ],
        compiler_params=pltpu.CompilerParams(
            dimension_semantics=("arbitrary", "arbitrary"),
        ),
        interpret=interpret,
    )(counts3d, C1, C2, C3)


def kernel(story, C0, C1, C2, C3):
    del C0
    story3d = story.reshape(_NTILES, _ROWS, _LANE)
    partial = _sc_counts(story3d)
    counts3d = (partial[0] + partial[1])[:_V].reshape(_NB, 1, _VB)
    return _tc_hops(counts3d, C1, C2, C3)
